# baseline (speedup 1.0000x reference)
import functools
import math

import jax
import jax.numpy as jnp
import numpy as np
from jax import lax
from jax.experimental import pallas as pl
from jax.experimental.pallas import tpu as pltpu
from jax.experimental.pallas import tpu_sc as plsc

_VOCAB = 1000000
_HIDDEN = 64
_BATCH = 4096
_SEQ = 200
_EPS = 1e-6

_NW = 32
_BB = _BATCH // _NW
_NCH = _HIDDEN // 16


def _pos_enc(seq_len, hidden_dim):
    position = np.arange(seq_len, dtype=np.float32)[:, None]
    div_term = np.exp(
        np.arange(0, hidden_dim, 2, dtype=np.float32)
        * (-math.log(10000.0) / hidden_dim)
    )
    pe = np.zeros((seq_len, hidden_dim), dtype=np.float32)
    pe[:, 0::2] = np.sin(position * div_term)
    pe[:, 1::2] = np.cos(position * div_term)
    return jnp.asarray(pe)


@functools.partial(
    pl.kernel,
    out_type=jax.ShapeDtypeStruct((_SEQ, 8, _NW, 1024), jnp.float32),
    mesh=plsc.VectorSubcoreMesh(core_axis_name="c", subcore_axis_name="s"),
    compiler_params=pltpu.CompilerParams(
        use_tc_tiling_on_sc=False, needs_layout_passes=False),
    scratch_types=[
        pltpu.VMEM((_SEQ, _HIDDEN), jnp.float32),
        pltpu.VMEM((_HIDDEN,), jnp.float32),
        pltpu.VMEM((_HIDDEN,), jnp.float32),
        pltpu.VMEM((_BB,), jnp.int32),
        pltpu.VMEM((_BB, _HIDDEN), jnp.float32),
        pltpu.VMEM((_HIDDEN * 128,), jnp.float32),
        pltpu.SemaphoreType.DMA,
    ],
)
def _emb_ln(xt_hbm, tab_hbm, pe_hbm, a_hbm, b_hbm, out_hbm,
            pe_v, a_v, b_v, idx_v, rows_v, img_v, sem):
    wid = lax.axis_index("s") * 2 + lax.axis_index("c")
    pltpu.sync_copy(pe_hbm, pe_v)
    pltpu.sync_copy(a_hbm, a_v)
    pltpu.sync_copy(b_hbm, b_v)
    a_c = [a_v[pl.ds(16 * c, 16)] for c in range(_NCH)]
    ab_c = [a_c[c] * b_v[pl.ds(16 * c, 16)] for c in range(_NCH)]

    iota = lax.iota(jnp.int32, 16)
    perms = [jnp.bitwise_xor(iota, np.int32(k)) for k in (1, 2, 4, 8)]
    flat_c = [(np.int32(16 * c) + iota) * np.int32(128) for c in range(_NCH)]

    def lane_sum(v):
        for p in perms:
            v = v + v.at[p].get(mode="promise_in_bounds")
        return v

    b0 = wid * _BB

    def unit(s, carry):
        pltpu.sync_copy(xt_hbm.at[s, pl.ds(b0, _BB)], idx_v)
        pltpu.async_copy(tab_hbm.at[idx_v], rows_v, sem).wait()
        p_c = [pe_v[s, pl.ds(16 * c, 16)] for c in range(_NCH)]

        def token(i, carry2):
            y = [rows_v[i, pl.ds(16 * c, 16)] + p_c[c] for c in range(_NCH)]
            sv = (y[0] + y[1]) + (y[2] + y[3])
            qv = (y[0] * y[0] + y[1] * y[1]) + (y[2] * y[2] + y[3] * y[3])
            ssum = lane_sum(sv)
            ssq = lane_sum(qv)
            mean = ssum * np.float32(1.0 / 64.0)
            var = (ssq - ssum * mean) * np.float32(1.0 / 63.0)
            var = jnp.maximum(var, np.float32(0.0))
            ii = lax.bitcast_convert_type(var, jnp.int32)
            ii = np.int32(0x5F3759DF) - lax.shift_right_arithmetic(ii, 1)
            r = lax.bitcast_convert_type(ii, jnp.float32)
            for _ in range(3):
                r = r * (np.float32(1.5) - np.float32(0.5) * var * r * r)
            sigma = var * r + np.float32(_EPS)
            inv = np.float32(1.0) / sigma
            for c in range(_NCH):
                o = (y[c] - mean) * inv * a_c[c] + ab_c[c]
                plsc.store_scatter(img_v, [flat_c[c] + i], o)
            return carry2

        lax.fori_loop(0, _BB, token, 0)
        for th in range(8):
            pltpu.async_copy(img_v.at[pl.ds(1024 * th, 1024)],
                             out_hbm.at[s, th, wid], sem).wait()
        return carry

    lax.fori_loop(0, _SEQ, unit, 0)


def kernel(x, emb_table, alpha, beta):
    pe = _pos_enc(_SEQ, _HIDDEN)
    out4 = _emb_ln(x.T, emb_table, pe, alpha, beta)
    out5 = out4.reshape(_SEQ, 8, _NW, 8, 128)
    return out5.transpose(2, 4, 0, 1, 3).reshape(_BATCH, _SEQ, _HIDDEN)


# speedup vs baseline: 1.1296x; 1.1296x over previous
import functools
import math

import jax
import jax.numpy as jnp
import numpy as np
from jax import lax
from jax.experimental import pallas as pl
from jax.experimental.pallas import tpu as pltpu
from jax.experimental.pallas import tpu_sc as plsc

_VOCAB = 1000000
_HIDDEN = 64
_BATCH = 4096
_SEQ = 200
_EPS = 1e-6

_NW = 32
_BB = _BATCH // _NW
_NCH = _HIDDEN // 16
_GRP = 4
_NGRP = _SEQ // _GRP


def _pos_enc(seq_len, hidden_dim):
    position = np.arange(seq_len, dtype=np.float32)[:, None]
    div_term = np.exp(
        np.arange(0, hidden_dim, 2, dtype=np.float32)
        * (-math.log(10000.0) / hidden_dim)
    )
    pe = np.zeros((seq_len, hidden_dim), dtype=np.float32)
    pe[:, 0::2] = np.sin(position * div_term)
    pe[:, 1::2] = np.cos(position * div_term)
    return jnp.asarray(pe)


@functools.partial(
    pl.kernel,
    out_type=jax.ShapeDtypeStruct((_SEQ, 8, _NW, 1024), jnp.float32),
    mesh=plsc.VectorSubcoreMesh(core_axis_name="c", subcore_axis_name="s"),
    compiler_params=pltpu.CompilerParams(
        use_tc_tiling_on_sc=False, needs_layout_passes=False),
    scratch_types=[
        pltpu.VMEM((_SEQ, _HIDDEN), jnp.float32),
        pltpu.VMEM((_HIDDEN,), jnp.float32),
        pltpu.VMEM((_HIDDEN,), jnp.float32),
        pltpu.VMEM((2 * _GRP, _BB), jnp.int32),
        pltpu.VMEM((_BB, _HIDDEN), jnp.float32),
        pltpu.VMEM((_BB, _HIDDEN), jnp.float32),
        pltpu.VMEM((_BB, _HIDDEN), jnp.float32),
        pltpu.VMEM((_BB, _HIDDEN), jnp.float32),
        pltpu.VMEM((_HIDDEN * 128,), jnp.float32),
        pltpu.VMEM((_HIDDEN * 128,), jnp.float32),
        pltpu.VMEM((_HIDDEN * 128,), jnp.float32),
        pltpu.VMEM((_HIDDEN * 128,), jnp.float32),
        pltpu.SemaphoreType.DMA,
        pltpu.SemaphoreType.DMA,
        pltpu.SemaphoreType.DMA,
        pltpu.SemaphoreType.DMA,
        pltpu.SemaphoreType.DMA,
        pltpu.SemaphoreType.DMA,
        pltpu.SemaphoreType.DMA,
    ],
)
def _emb_ln(xt_hbm, tab_hbm, pe_hbm, a_hbm, b_hbm, out_hbm,
            pe_v, a_v, b_v, idx_v, r0, r1, r2, r3, m0, m1, m2, m3,
            isem0, isem1, gsem, os0, os1, os2, os3):
    rows = (r0, r1, r2, r3)
    imgs = (m0, m1, m2, m3)
    osems = (os0, os1, os2, os3)
    isems = (isem0, isem1)

    wid = lax.axis_index("s") * 2 + lax.axis_index("c")
    pltpu.sync_copy(pe_hbm, pe_v)
    pltpu.sync_copy(a_hbm, a_v)
    pltpu.sync_copy(b_hbm, b_v)
    a_c = [a_v[pl.ds(16 * c, 16)] for c in range(_NCH)]
    ab_c = [a_c[c] * b_v[pl.ds(16 * c, 16)] for c in range(_NCH)]

    iota = lax.iota(jnp.int32, 16)
    perms = [jnp.bitwise_xor(iota, np.int32(k)) for k in (1, 2, 4, 8)]
    flat_c = [(np.int32(16 * c) + iota) * np.int32(128) for c in range(_NCH)]

    def lane_sum(v):
        for p in perms:
            v = v + v.at[p].get(mode="promise_in_bounds")
        return v

    b0 = wid * _BB

    def fire_idx(s, slot, sem):
        return pltpu.async_copy(
            xt_hbm.at[s, pl.ds(b0, _BB)], idx_v.at[slot], sem)

    def make_token(rows_b, img_b, p_c):
        def token(i, carry2):
            y = [rows_b[i, pl.ds(16 * c, 16)] + p_c[c] for c in range(_NCH)]
            sv = (y[0] + y[1]) + (y[2] + y[3])
            qv = (y[0] * y[0] + y[1] * y[1]) + (y[2] * y[2] + y[3] * y[3])
            ssum = lane_sum(sv)
            ssq = lane_sum(qv)
            mean = ssum * np.float32(1.0 / 64.0)
            var = (ssq - ssum * mean) * np.float32(1.0 / 63.0)
            var = jnp.maximum(var, np.float32(0.0))
            ii = lax.bitcast_convert_type(var, jnp.int32)
            ii = np.int32(0x5F3759DF) - lax.shift_right_arithmetic(ii, 1)
            r = lax.bitcast_convert_type(ii, jnp.float32)
            for _ in range(3):
                r = r * (np.float32(1.5) - np.float32(0.5) * var * r * r)
            sigma = var * r + np.float32(_EPS)
            inv = np.float32(1.0) / sigma
            for c in range(_NCH):
                o = (y[c] - mean) * inv * a_c[c] + ab_c[c]
                plsc.store_scatter(img_b, [flat_c[c] + i], o)
            return carry2

        return token

    for b in range(_GRP):
        fire_idx(b, b, isem0)

    @pl.loop(0, _NGRP // 2)
    def group_pair(j):
        for sub in range(2):
            s4 = j * 2 * _GRP + sub * _GRP
            half = sub * _GRP
            nexthalf = (1 - sub) * _GRP

            @pl.when(s4 + _GRP < _SEQ)
            def _():
                for b in range(_GRP):
                    fire_idx(s4 + _GRP + b, nexthalf + b, isems[1 - sub])

            for b in range(_GRP):
                pltpu.make_async_copy(
                    xt_hbm.at[s4 + b, pl.ds(b0, _BB)],
                    idx_v.at[half + b], isems[sub]).wait()
            ghandles = [
                pltpu.async_copy(tab_hbm.at[idx_v.at[half + b]],
                                 rows[b], gsem)
                for b in range(_GRP)
            ]
            for b in range(_GRP):
                s = s4 + b
                ghandles[b].wait()
                @pl.when(s4 > 0)
                def _():
                    for th in range(8):
                        pltpu.make_async_copy(
                            imgs[b].at[pl.ds(1024 * th, 1024)],
                            out_hbm.at[s, th, wid], osems[b]).wait()
                p_c = [pe_v[s, pl.ds(16 * c, 16)] for c in range(_NCH)]
                lax.fori_loop(0, _BB, make_token(rows[b], imgs[b], p_c), 0)
                for th in range(8):
                    pltpu.async_copy(imgs[b].at[pl.ds(1024 * th, 1024)],
                                     out_hbm.at[s, th, wid], osems[b])

    for b in range(_GRP):
        for th in range(8):
            pltpu.make_async_copy(
                imgs[b].at[pl.ds(1024 * th, 1024)],
                out_hbm.at[_SEQ - _GRP + b, th, wid], osems[b]).wait()


def kernel(x, emb_table, alpha, beta):
    pe = _pos_enc(_SEQ, _HIDDEN)
    out4 = _emb_ln(x.T, emb_table, pe, alpha, beta)
    out5 = out4.reshape(_SEQ, 8, _NW, 8, 128)
    return out5.transpose(2, 4, 0, 1, 3).reshape(_BATCH, _SEQ, _HIDDEN)


# speedup vs baseline: 1.2368x; 1.0949x over previous
import functools
import math

import jax
import jax.numpy as jnp
import numpy as np
from jax import lax
from jax.experimental import pallas as pl
from jax.experimental.pallas import tpu as pltpu
from jax.experimental.pallas import tpu_sc as plsc

_VOCAB = 1000000
_HIDDEN = 64
_BATCH = 4096
_SEQ = 200
_EPS = 1e-6

_NW = 32
_BB = _BATCH // _NW
_NCH = _HIDDEN // 16
_GRP = 4
_NGRP = _SEQ // _GRP


def _pos_enc(seq_len, hidden_dim):
    position = np.arange(seq_len, dtype=np.float32)[:, None]
    div_term = np.exp(
        np.arange(0, hidden_dim, 2, dtype=np.float32)
        * (-math.log(10000.0) / hidden_dim)
    )
    pe = np.zeros((seq_len, hidden_dim), dtype=np.float32)
    pe[:, 0::2] = np.sin(position * div_term)
    pe[:, 1::2] = np.cos(position * div_term)
    return jnp.asarray(pe)


@functools.partial(
    pl.kernel,
    out_type=jax.ShapeDtypeStruct((_SEQ, 8, _NW, 1024), jnp.float32),
    mesh=plsc.VectorSubcoreMesh(core_axis_name="c", subcore_axis_name="s"),
    compiler_params=pltpu.CompilerParams(
        use_tc_tiling_on_sc=False, needs_layout_passes=False),
    scratch_types=[
        pltpu.VMEM((_SEQ, _HIDDEN), jnp.float32),
        pltpu.VMEM((_HIDDEN,), jnp.float32),
        pltpu.VMEM((_HIDDEN,), jnp.float32),
        pltpu.VMEM((2 * _GRP, _BB), jnp.int32),
        pltpu.VMEM((_BB, _HIDDEN), jnp.float32),
        pltpu.VMEM((_BB, _HIDDEN), jnp.float32),
        pltpu.VMEM((_BB, _HIDDEN), jnp.float32),
        pltpu.VMEM((_BB, _HIDDEN), jnp.float32),
        pltpu.VMEM((_HIDDEN * 128,), jnp.float32),
        pltpu.VMEM((_HIDDEN * 128,), jnp.float32),
        pltpu.VMEM((_HIDDEN * 128,), jnp.float32),
        pltpu.VMEM((_HIDDEN * 128,), jnp.float32),
        pltpu.SemaphoreType.DMA,
        pltpu.SemaphoreType.DMA,
        pltpu.SemaphoreType.DMA,
        pltpu.SemaphoreType.DMA,
        pltpu.SemaphoreType.DMA,
        pltpu.SemaphoreType.DMA,
        pltpu.SemaphoreType.DMA,
    ],
)
def _emb_ln(xt_hbm, tab_hbm, pe_hbm, a_hbm, b_hbm, out_hbm,
            pe_v, a_v, b_v, idx_v, r0, r1, r2, r3, m0, m1, m2, m3,
            isem0, isem1, gsem, os0, os1, os2, os3):
    rows = (r0, r1, r2, r3)
    imgs = (m0, m1, m2, m3)
    osems = (os0, os1, os2, os3)
    isems = (isem0, isem1)

    wid = lax.axis_index("s") * 2 + lax.axis_index("c")
    pltpu.sync_copy(pe_hbm, pe_v)
    pltpu.sync_copy(a_hbm, a_v)
    pltpu.sync_copy(b_hbm, b_v)
    a_c = [a_v[pl.ds(16 * c, 16)] for c in range(_NCH)]
    ab_c = [a_c[c] * b_v[pl.ds(16 * c, 16)] for c in range(_NCH)]

    iota = lax.iota(jnp.int32, 16)
    perms = [jnp.bitwise_xor(iota, np.int32(k)) for k in (1, 2, 4, 8)]
    flat_c = [(np.int32(16 * c) + iota) * np.int32(128) for c in range(_NCH)]

    def lane_sum(v):
        for p in perms:
            v = v + v.at[p].get(mode="promise_in_bounds")
        return v

    b0 = wid * _BB

    def fire_idx(s, slot, sem):
        return pltpu.async_copy(
            xt_hbm.at[s, pl.ds(b0, _BB)], idx_v.at[slot], sem)

    def make_token(rows_b, img_b, p_c):
        def token(i, carry2):
            y = [rows_b[i, pl.ds(16 * c, 16)] + p_c[c] for c in range(_NCH)]
            sv = (y[0] + y[1]) + (y[2] + y[3])
            qv = (y[0] * y[0] + y[1] * y[1]) + (y[2] * y[2] + y[3] * y[3])
            ssum = lane_sum(sv)
            ssq = lane_sum(qv)
            mean = ssum * np.float32(1.0 / 64.0)
            var = (ssq - ssum * mean) * np.float32(1.0 / 63.0)
            var = jnp.maximum(var, np.float32(1e-6))
            ii = lax.bitcast_convert_type(var, jnp.int32)
            ii = np.int32(0x5F3759DF) - lax.shift_right_arithmetic(ii, 1)
            r = lax.bitcast_convert_type(ii, jnp.float32)
            for _ in range(2):
                r = r * (np.float32(1.5) - np.float32(0.5) * var * r * r)
            inv = r - np.float32(_EPS) * (r * r)
            for c in range(_NCH):
                o = (y[c] - mean) * inv * a_c[c] + ab_c[c]
                plsc.store_scatter(img_b, [flat_c[c] + i], o)
            return carry2

        return token

    for b in range(_GRP):
        fire_idx(b, b, isem0)

    @pl.loop(0, _NGRP // 2)
    def group_pair(j):
        for sub in range(2):
            s4 = j * 2 * _GRP + sub * _GRP
            half = sub * _GRP
            nexthalf = (1 - sub) * _GRP

            @pl.when(s4 + _GRP < _SEQ)
            def _():
                for b in range(_GRP):
                    fire_idx(s4 + _GRP + b, nexthalf + b, isems[1 - sub])

            for b in range(_GRP):
                pltpu.make_async_copy(
                    xt_hbm.at[s4 + b, pl.ds(b0, _BB)],
                    idx_v.at[half + b], isems[sub]).wait()
            ghandles = [
                pltpu.async_copy(tab_hbm.at[idx_v.at[half + b]],
                                 rows[b], gsem)
                for b in range(_GRP)
            ]
            for b in range(_GRP):
                s = s4 + b
                ghandles[b].wait()
                @pl.when(s4 > 0)
                def _():
                    for th in range(8):
                        pltpu.make_async_copy(
                            imgs[b].at[pl.ds(1024 * th, 1024)],
                            out_hbm.at[s, th, wid], osems[b]).wait()
                p_c = [pe_v[s, pl.ds(16 * c, 16)] for c in range(_NCH)]
                lax.fori_loop(0, _BB, make_token(rows[b], imgs[b], p_c), 0,
                              unroll=4)
                for th in range(8):
                    pltpu.async_copy(imgs[b].at[pl.ds(1024 * th, 1024)],
                                     out_hbm.at[s, th, wid], osems[b])

    for b in range(_GRP):
        for th in range(8):
            pltpu.make_async_copy(
                imgs[b].at[pl.ds(1024 * th, 1024)],
                out_hbm.at[_SEQ - _GRP + b, th, wid], osems[b]).wait()


def kernel(x, emb_table, alpha, beta):
    pe = _pos_enc(_SEQ, _HIDDEN)
    out4 = _emb_ln(x.T, emb_table, pe, alpha, beta)
    out5 = out4.reshape(_SEQ, 8, _NW, 8, 128)
    return out5.transpose(2, 4, 0, 1, 3).reshape(_BATCH, _SEQ, _HIDDEN)


# speedup vs baseline: 1.7978x; 1.4536x over previous
import functools
import math

import jax
import jax.numpy as jnp
import numpy as np
from jax import lax
from jax.experimental import pallas as pl
from jax.experimental.pallas import tpu as pltpu
from jax.experimental.pallas import tpu_sc as plsc

_VOCAB = 1000000
_HIDDEN = 64
_BATCH = 4096
_SEQ = 200
_EPS = 1e-6

_NW = 32
_BB = _BATCH // _NW
_NCH = _HIDDEN // 16
_GRP = 4
_NGRP = _SEQ // _GRP


def _pos_enc(seq_len, hidden_dim):
    position = np.arange(seq_len, dtype=np.float32)[:, None]
    div_term = np.exp(
        np.arange(0, hidden_dim, 2, dtype=np.float32)
        * (-math.log(10000.0) / hidden_dim)
    )
    pe = np.zeros((seq_len, hidden_dim), dtype=np.float32)
    pe[:, 0::2] = np.sin(position * div_term)
    pe[:, 1::2] = np.cos(position * div_term)
    return jnp.asarray(pe)


@functools.partial(
    pl.kernel,
    out_type=jax.ShapeDtypeStruct((_SEQ, 8, _NW, 1024), jnp.float32),
    mesh=plsc.VectorSubcoreMesh(core_axis_name="c", subcore_axis_name="s"),
    compiler_params=pltpu.CompilerParams(
        use_tc_tiling_on_sc=False, needs_layout_passes=False),
    scratch_types=[
        pltpu.VMEM((_SEQ, _HIDDEN), jnp.float32),
        pltpu.VMEM((_HIDDEN,), jnp.float32),
        pltpu.VMEM((_HIDDEN,), jnp.float32),
        pltpu.VMEM((2 * _GRP, _BB), jnp.int32),
        pltpu.VMEM((_BB, _HIDDEN), jnp.float32),
        pltpu.VMEM((_BB, _HIDDEN), jnp.float32),
        pltpu.VMEM((_BB, _HIDDEN), jnp.float32),
        pltpu.VMEM((_BB, _HIDDEN), jnp.float32),
        pltpu.VMEM((_HIDDEN * 128,), jnp.float32),
        pltpu.VMEM((_HIDDEN * 128,), jnp.float32),
        pltpu.VMEM((_HIDDEN * 128,), jnp.float32),
        pltpu.VMEM((_HIDDEN * 128,), jnp.float32),
        pltpu.SemaphoreType.DMA,
        pltpu.SemaphoreType.DMA,
        pltpu.SemaphoreType.DMA,
        pltpu.SemaphoreType.DMA,
        pltpu.SemaphoreType.DMA,
        pltpu.SemaphoreType.DMA,
        pltpu.SemaphoreType.DMA,
    ],
)
def _emb_ln(xt_hbm, tab_hbm, pe_hbm, a_hbm, b_hbm, out_hbm,
            pe_v, a_v, b_v, idx_v, r0, r1, r2, r3, m0, m1, m2, m3,
            isem0, isem1, gsem, os0, os1, os2, os3):
    rows = (r0, r1, r2, r3)
    imgs = (m0, m1, m2, m3)
    osems = (os0, os1, os2, os3)
    isems = (isem0, isem1)

    wid = lax.axis_index("s") * 2 + lax.axis_index("c")
    pltpu.sync_copy(pe_hbm, pe_v)
    pltpu.sync_copy(a_hbm, a_v)
    pltpu.sync_copy(b_hbm, b_v)
    a_c = [a_v[pl.ds(16 * c, 16)] for c in range(_NCH)]
    ab_c = [a_c[c] * b_v[pl.ds(16 * c, 16)] for c in range(_NCH)]

    iota = lax.iota(jnp.int32, 16)
    perms = [jnp.bitwise_xor(iota, np.int32(k)) for k in (1, 2, 4, 8)]
    flat_c = [(np.int32(16 * c) + iota) * np.int32(128) for c in range(_NCH)]

    def lane_sum(v):
        for p in perms:
            v = v + v.at[p].get(mode="promise_in_bounds")
        return v

    b0 = wid * _BB

    def fire_idx(s, slot, sem):
        return pltpu.async_copy(
            xt_hbm.at[s, pl.ds(b0, _BB)], idx_v.at[slot], sem)

    def make_token(rows_b, img_b, p_c):
        def token(i, carry2):
            y = [rows_b[i, pl.ds(16 * c, 16)] + p_c[c] for c in range(_NCH)]
            sv = (y[0] + y[1]) + (y[2] + y[3])
            qv = (y[0] * y[0] + y[1] * y[1]) + (y[2] * y[2] + y[3] * y[3])
            ssum = lane_sum(sv)
            ssq = lane_sum(qv)
            mean = ssum * np.float32(1.0 / 64.0)
            var = (ssq - ssum * mean) * np.float32(1.0 / 63.0)
            var = jnp.maximum(var, np.float32(1e-6))
            ii = lax.bitcast_convert_type(var, jnp.int32)
            ii = np.int32(0x5F3759DF) - lax.shift_right_arithmetic(ii, 1)
            r = lax.bitcast_convert_type(ii, jnp.float32)
            for _ in range(2):
                r = r * (np.float32(1.5) - np.float32(0.5) * var * r * r)
            inv = r - np.float32(_EPS) * (r * r)
            for c in range(_NCH):
                o = (y[c] - mean) * inv * a_c[c] + ab_c[c]
                plsc.store_scatter(img_b, [flat_c[c] + i], o)
            return carry2

        return token

    for b in range(_GRP):
        fire_idx(b, b, isem0)

    @pl.loop(0, _NGRP // 2)
    def group_pair(j):
        for sub in range(2):
            s4 = j * 2 * _GRP + sub * _GRP
            half = sub * _GRP
            nexthalf = (1 - sub) * _GRP

            @pl.when(s4 + _GRP < _SEQ)
            def _():
                for b in range(_GRP):
                    fire_idx(s4 + _GRP + b, nexthalf + b, isems[1 - sub])

            for b in range(_GRP):
                pltpu.make_async_copy(
                    xt_hbm.at[s4 + b, pl.ds(b0, _BB)],
                    idx_v.at[half + b], isems[sub]).wait()
            ghandles = [
                pltpu.async_copy(tab_hbm.at[idx_v.at[half + b]],
                                 rows[b], gsem)
                for b in range(_GRP)
            ]
            for b in range(_GRP):
                s = s4 + b
                ghandles[b].wait()
                @pl.when(s4 > 0)
                def _():
                    for th in range(8):
                        pltpu.make_async_copy(
                            imgs[b].at[pl.ds(1024 * th, 1024)],
                            out_hbm.at[s, th, wid], osems[b]).wait()
                p_c = [pe_v[s, pl.ds(16 * c, 16)] for c in range(_NCH)]
                tok = make_token(rows[b], imgs[b], p_c)
                plsc.parallel_loop(0, _BB, unroll=4)(
                    lambda i, _tok=tok: _tok(i, 0) and None)
                for th in range(8):
                    pltpu.async_copy(imgs[b].at[pl.ds(1024 * th, 1024)],
                                     out_hbm.at[s, th, wid], osems[b])

    for b in range(_GRP):
        for th in range(8):
            pltpu.make_async_copy(
                imgs[b].at[pl.ds(1024 * th, 1024)],
                out_hbm.at[_SEQ - _GRP + b, th, wid], osems[b]).wait()


def kernel(x, emb_table, alpha, beta):
    pe = _pos_enc(_SEQ, _HIDDEN)
    out4 = _emb_ln(x.T, emb_table, pe, alpha, beta)
    out5 = out4.reshape(_SEQ, 8, _NW, 8, 128)
    return out5.transpose(2, 4, 0, 1, 3).reshape(_BATCH, _SEQ, _HIDDEN)
